# baseline (speedup 1.0000x reference)
import functools

import jax
import jax.numpy as jnp
from jax import lax
from jax.experimental import pallas as pl
from jax.experimental.pallas import tpu as pltpu
from jax.experimental.pallas import tpu_sc as plsc

NC = 2
NS = 16
NW = NC * NS
LANES = 16


def _make_sc_gather(n_nodes, n_edges):
    ept = n_edges // NW
    ch = 6272
    assert ept % ch == 0 and ch % LANES == 0
    mesh = plsc.VectorSubcoreMesh(core_axis_name="c", subcore_axis_name="s")

    def body(h_hbm, esrc_hbm, edst_hbm, hs_hbm, hd_hbm,
             table, sbuf, dbuf, hsb, hdb):
        c = lax.axis_index("c")
        s = lax.axis_index("s")
        base = (s * NC + c) * ept
        pltpu.sync_copy(h_hbm, table)

        def chunk_body(ci, carry):
            off = base + ci * ch
            pltpu.sync_copy(esrc_hbm.at[pl.ds(off, ch)], sbuf)
            pltpu.sync_copy(edst_hbm.at[pl.ds(off, ch)], dbuf)

            def vec_body(k, carry2):
                i0 = k * LANES
                si = sbuf[pl.ds(i0, LANES)]
                di = dbuf[pl.ds(i0, LANES)]
                hsb[pl.ds(i0, LANES)] = plsc.load_gather(table, [si])
                hdb[pl.ds(i0, LANES)] = plsc.load_gather(table, [di])
                return carry2

            lax.fori_loop(0, ch // LANES, vec_body, 0, unroll=4)
            pltpu.sync_copy(hsb, hs_hbm.at[pl.ds(off, ch)])
            pltpu.sync_copy(hdb, hd_hbm.at[pl.ds(off, ch)])
            return carry

        lax.fori_loop(0, ept // ch, chunk_body, 0)

    return pl.kernel(
        body,
        out_type=[jax.ShapeDtypeStruct((n_edges,), jnp.float32),
                  jax.ShapeDtypeStruct((n_edges,), jnp.float32)],
        mesh=mesh,
        compiler_params=pltpu.CompilerParams(needs_layout_passes=False),
        scratch_types=[
            pltpu.VMEM((n_nodes,), jnp.float32),
            pltpu.VMEM((ch,), jnp.int32),
            pltpu.VMEM((ch,), jnp.int32),
            pltpu.VMEM((ch,), jnp.float32),
            pltpu.VMEM((ch,), jnp.float32),
        ],
    )


def _mlp_body(w1_ref, b1_ref, w2_ref, b2_ref, hs_ref, hd_ref, w_ref, out_ref):
    hs = hs_ref[...]
    hd = hd_ref[...]
    w = w_ref[...]
    hidden = w1_ref.shape[1]

    def jb(j, acc):
        t = hs * w1_ref[0, j] + hd * w1_ref[1, j] + w * w1_ref[2, j] + b1_ref[j]
        return acc + jax.nn.gelu(t) * w2_ref[j]

    acc = lax.fori_loop(0, hidden, jb, jnp.zeros_like(hs))
    out_ref[...] = (acc + b2_ref[0]) * w


def _make_tc_mlp(rows, hidden):
    rb = 784
    assert rows % rb == 0
    grid = (rows // rb,)
    smem = functools.partial(pl.BlockSpec, memory_space=pltpu.SMEM)
    dblk = pl.BlockSpec((rb, 128), lambda i: (i, 0))
    return pl.pallas_call(
        _mlp_body,
        grid=grid,
        in_specs=[smem((3, hidden), lambda i: (0, 0)),
                  smem((hidden,), lambda i: (0,)),
                  smem((hidden,), lambda i: (0,)),
                  smem((1,), lambda i: (0,)),
                  dblk, dblk, dblk],
        out_specs=dblk,
        out_shape=jax.ShapeDtypeStruct((rows, 128), jnp.float32),
    )


def _make_sc_scatter(n_nodes, rows):
    rpt = rows // NW
    kr = 56
    assert rpt % kr == 0
    nps = n_nodes // NS
    assert nps % LANES == 0
    mesh = plsc.VectorSubcoreMesh(core_axis_name="c", subcore_axis_name="s")

    def body(msg_hbm, edst_hbm, parts_hbm, idxb, valb, accl, acc_sh):
        c = lax.axis_index("c")
        s = lax.axis_index("s")
        wid = s * NC + c

        def zb(k, carry):
            accl[pl.ds(k * LANES, LANES)] = jnp.zeros((LANES,), jnp.float32)
            return carry

        lax.fori_loop(0, nps // LANES, zb, 0, unroll=8)
        pltpu.sync_copy(accl, acc_sh.at[pl.ds(s * nps, nps)])
        plsc.subcore_barrier()

        row0 = wid * rpt

        def chunk_body(ci, carry):
            r0 = row0 + ci * kr
            pltpu.sync_copy(edst_hbm.at[pl.ds(r0, kr), :], idxb)
            pltpu.sync_copy(msg_hbm.at[pl.ds(r0, kr), :], valb)

            def rower(j, carry2):
                pltpu.sync_copy(valb.at[j], acc_sh.at[idxb.at[j]], add=True)
                return carry2

            lax.fori_loop(0, kr, rower, 0)
            return carry

        lax.fori_loop(0, rpt // kr, chunk_body, 0)
        plsc.subcore_barrier()

        pltpu.sync_copy(acc_sh.at[pl.ds(s * nps, nps)], accl)
        pltpu.sync_copy(accl, parts_hbm.at[pl.ds(c * n_nodes + s * nps, nps)])

    return pl.kernel(
        body,
        out_type=jax.ShapeDtypeStruct((NC * n_nodes,), jnp.float32),
        mesh=mesh,
        compiler_params=pltpu.CompilerParams(needs_layout_passes=False),
        scratch_types=[
            pltpu.VMEM((kr, 128), jnp.int32),
            pltpu.VMEM((kr, 128), jnp.float32),
            pltpu.VMEM((nps,), jnp.float32),
            pltpu.VMEM_SHARED((n_nodes,), jnp.float32),
        ],
    )


def _combine_body(softplus, lnp_ref, h_ref, p_ref, deg_ref, out_ref):
    h = h_ref[...]
    agg = (p_ref[0] + p_ref[1]) / deg_ref[...]
    x = h + agg
    mu = x
    var = (x - mu) * (x - mu)
    hn = (x - mu) / jnp.sqrt(var + 1e-6) * lnp_ref[0] + lnp_ref[1]
    out_ref[...] = jax.nn.softplus(hn) if softplus else hn


def _make_tc_combine(nrows, softplus):
    full = pl.BlockSpec((nrows, 128), lambda: (0, 0))
    return pl.pallas_call(
        functools.partial(_combine_body, softplus),
        in_specs=[pl.BlockSpec(memory_space=pltpu.SMEM),
                  full,
                  pl.BlockSpec((2, nrows, 128), lambda: (0, 0, 0)),
                  full],
        out_specs=full,
        out_shape=jax.ShapeDtypeStruct((nrows, 128), jnp.float32),
    )


def kernel(eps_2d, esrc, edst, ew, ndeg, W1, b1, W2, b2, ln_scale, ln_bias):
    n_nodes = eps_2d.shape[0] * eps_2d.shape[1]
    n_edges = esrc.shape[0]
    n_layers, _, hidden = W1.shape
    rows = n_edges // 128
    nrows = n_nodes // 128

    h = eps_2d.reshape((n_nodes,))
    edst2d = edst.reshape((rows, 128))
    ew2d = ew.reshape((rows, 128))
    deg2d = ndeg.reshape((nrows, 128))

    sc_gather = _make_sc_gather(n_nodes, n_edges)
    tc_mlp = _make_tc_mlp(rows, hidden)
    sc_scatter = _make_sc_scatter(n_nodes, rows)

    for i in range(n_layers):
        hs, hd = sc_gather(h, esrc, edst)
        msgw = tc_mlp(W1[i], b1[i], W2[i, :, 0], b2[i],
                      hs.reshape((rows, 128)), hd.reshape((rows, 128)), ew2d)
        parts = sc_scatter(msgw, edst2d)
        lnp = jnp.stack([ln_scale[i, 0], ln_bias[i, 0]])
        combine = _make_tc_combine(nrows, softplus=(i == n_layers - 1))
        h2d = combine(lnp, h.reshape((nrows, 128)),
                      parts.reshape((2, nrows, 128)), deg2d)
        h = h2d.reshape((n_nodes,))

    return h.reshape(eps_2d.shape)


# speedup vs baseline: 1.0164x; 1.0164x over previous
import functools

import jax
import jax.numpy as jnp
from jax import lax
from jax.experimental import pallas as pl
from jax.experimental.pallas import tpu as pltpu
from jax.experimental.pallas import tpu_sc as plsc

NC = 2
NS = 16
NW = NC * NS
LANES = 16


def _make_sc_gather(n_nodes, n_edges):
    ept = n_edges // NW
    ch = 6272
    assert ept % ch == 0 and ch % LANES == 0
    mesh = plsc.VectorSubcoreMesh(core_axis_name="c", subcore_axis_name="s")

    def body(h_hbm, esrc_hbm, edst_hbm, hs_hbm, hd_hbm,
             table, sbuf, dbuf, hsb, hdb):
        c = lax.axis_index("c")
        s = lax.axis_index("s")
        base = (s * NC + c) * ept
        pltpu.sync_copy(h_hbm, table)

        def chunk_body(ci, carry):
            off = base + ci * ch
            pltpu.sync_copy(esrc_hbm.at[pl.ds(off, ch)], sbuf)
            pltpu.sync_copy(edst_hbm.at[pl.ds(off, ch)], dbuf)

            def vec_body(k, carry2):
                i0 = k * LANES
                si = sbuf[pl.ds(i0, LANES)]
                di = dbuf[pl.ds(i0, LANES)]
                hsb[pl.ds(i0, LANES)] = plsc.load_gather(table, [si])
                hdb[pl.ds(i0, LANES)] = plsc.load_gather(table, [di])
                return carry2

            lax.fori_loop(0, ch // LANES, vec_body, 0, unroll=4)
            pltpu.sync_copy(hsb, hs_hbm.at[pl.ds(off, ch)])
            pltpu.sync_copy(hdb, hd_hbm.at[pl.ds(off, ch)])
            return carry

        lax.fori_loop(0, ept // ch, chunk_body, 0)

    return pl.kernel(
        body,
        out_type=[jax.ShapeDtypeStruct((n_edges,), jnp.float32),
                  jax.ShapeDtypeStruct((n_edges,), jnp.float32)],
        mesh=mesh,
        compiler_params=pltpu.CompilerParams(needs_layout_passes=False),
        scratch_types=[
            pltpu.VMEM((n_nodes,), jnp.float32),
            pltpu.VMEM((ch,), jnp.int32),
            pltpu.VMEM((ch,), jnp.int32),
            pltpu.VMEM((ch,), jnp.float32),
            pltpu.VMEM((ch,), jnp.float32),
        ],
    )


def _mlp_body(rc, w1_ref, b1_ref, w2_ref, b2_ref, hs_ref, hd_ref, w_ref,
              out_ref):
    hidden = w1_ref.shape[1]
    rows = hs_ref.shape[0]

    def chunk(k, carry):
        sl = pl.ds(k * rc, rc)
        hs = hs_ref[sl, :]
        hd = hd_ref[sl, :]
        w = w_ref[sl, :]

        def jb(j, acc):
            t = (hs * w1_ref[0, j] + hd * w1_ref[1, j] + w * w1_ref[2, j]
                 + b1_ref[j])
            return acc + jax.nn.gelu(t) * w2_ref[j]

        acc = lax.fori_loop(0, hidden, jb,
                            jnp.zeros((rc, 128), jnp.float32))
        out_ref[sl, :] = (acc + b2_ref[0]) * w
        return carry

    lax.fori_loop(0, rows // rc, chunk, 0)


def _make_tc_mlp(rows, hidden):
    rb = 784
    rc = 56
    assert rows % rb == 0 and rb % rc == 0
    grid = (rows // rb,)
    smem = functools.partial(pl.BlockSpec, memory_space=pltpu.SMEM)
    dblk = pl.BlockSpec((rb, 128), lambda i: (i, 0))
    return pl.pallas_call(
        functools.partial(_mlp_body, rc),
        grid=grid,
        in_specs=[smem((3, hidden), lambda i: (0, 0)),
                  smem((hidden,), lambda i: (0,)),
                  smem((hidden,), lambda i: (0,)),
                  smem((1,), lambda i: (0,)),
                  dblk, dblk, dblk],
        out_specs=dblk,
        out_shape=jax.ShapeDtypeStruct((rows, 128), jnp.float32),
    )


def _make_sc_scatter(n_nodes, rows):
    rpt = rows // NW
    kr = 56
    assert rpt % kr == 0
    nps = n_nodes // NS
    assert nps % LANES == 0
    mesh = plsc.VectorSubcoreMesh(core_axis_name="c", subcore_axis_name="s")

    def body(msg_hbm, edst_hbm, parts_hbm, idxb, valb, accl, acc_sh):
        c = lax.axis_index("c")
        s = lax.axis_index("s")
        wid = s * NC + c

        def zb(k, carry):
            accl[pl.ds(k * LANES, LANES)] = jnp.zeros((LANES,), jnp.float32)
            return carry

        lax.fori_loop(0, nps // LANES, zb, 0, unroll=8)
        pltpu.sync_copy(accl, acc_sh.at[pl.ds(s * nps, nps)])
        plsc.subcore_barrier()

        row0 = wid * rpt

        def chunk_body(ci, carry):
            r0 = row0 + ci * kr
            pltpu.sync_copy(edst_hbm.at[pl.ds(r0, kr), :], idxb)
            pltpu.sync_copy(msg_hbm.at[pl.ds(r0, kr), :], valb)

            def rower(j, carry2):
                pltpu.sync_copy(valb.at[j], acc_sh.at[idxb.at[j]], add=True)
                return carry2

            lax.fori_loop(0, kr, rower, 0)
            return carry

        lax.fori_loop(0, rpt // kr, chunk_body, 0)
        plsc.subcore_barrier()

        pltpu.sync_copy(acc_sh.at[pl.ds(s * nps, nps)], accl)
        pltpu.sync_copy(accl, parts_hbm.at[pl.ds(c * n_nodes + s * nps, nps)])

    return pl.kernel(
        body,
        out_type=jax.ShapeDtypeStruct((NC * n_nodes,), jnp.float32),
        mesh=mesh,
        compiler_params=pltpu.CompilerParams(needs_layout_passes=False),
        scratch_types=[
            pltpu.VMEM((kr, 128), jnp.int32),
            pltpu.VMEM((kr, 128), jnp.float32),
            pltpu.VMEM((nps,), jnp.float32),
            pltpu.VMEM_SHARED((n_nodes,), jnp.float32),
        ],
    )


def _combine_body(softplus, lnp_ref, h_ref, p_ref, deg_ref, out_ref):
    h = h_ref[...]
    agg = (p_ref[0] + p_ref[1]) / deg_ref[...]
    x = h + agg
    mu = x
    var = (x - mu) * (x - mu)
    hn = (x - mu) / jnp.sqrt(var + 1e-6) * lnp_ref[0] + lnp_ref[1]
    out_ref[...] = jax.nn.softplus(hn) if softplus else hn


def _make_tc_combine(nrows, softplus):
    full = pl.BlockSpec((nrows, 128), lambda: (0, 0))
    return pl.pallas_call(
        functools.partial(_combine_body, softplus),
        in_specs=[pl.BlockSpec(memory_space=pltpu.SMEM),
                  full,
                  pl.BlockSpec((2, nrows, 128), lambda: (0, 0, 0)),
                  full],
        out_specs=full,
        out_shape=jax.ShapeDtypeStruct((nrows, 128), jnp.float32),
    )


def kernel(eps_2d, esrc, edst, ew, ndeg, W1, b1, W2, b2, ln_scale, ln_bias):
    n_nodes = eps_2d.shape[0] * eps_2d.shape[1]
    n_edges = esrc.shape[0]
    n_layers, _, hidden = W1.shape
    rows = n_edges // 128
    nrows = n_nodes // 128

    h = eps_2d.reshape((n_nodes,))
    edst2d = edst.reshape((rows, 128))
    ew2d = ew.reshape((rows, 128))
    deg2d = ndeg.reshape((nrows, 128))

    sc_gather = _make_sc_gather(n_nodes, n_edges)
    tc_mlp = _make_tc_mlp(rows, hidden)
    sc_scatter = _make_sc_scatter(n_nodes, rows)

    for i in range(n_layers):
        hs, hd = sc_gather(h, esrc, edst)
        msgw = tc_mlp(W1[i], b1[i], W2[i, :, 0], b2[i],
                      hs.reshape((rows, 128)), hd.reshape((rows, 128)), ew2d)
        parts = sc_scatter(msgw, edst2d)
        lnp = jnp.stack([ln_scale[i, 0], ln_bias[i, 0]])
        combine = _make_tc_combine(nrows, softplus=(i == n_layers - 1))
        h2d = combine(lnp, h.reshape((nrows, 128)),
                      parts.reshape((2, nrows, 128)), deg2d)
        h = h2d.reshape((n_nodes,))

    return h.reshape(eps_2d.shape)


# speedup vs baseline: 1.3264x; 1.3050x over previous
import functools

import jax
import jax.numpy as jnp
from jax import lax
from jax.experimental import pallas as pl
from jax.experimental.pallas import tpu as pltpu
from jax.experimental.pallas import tpu_sc as plsc

NC = 2
NS = 16
NW = NC * NS
LANES = 16


def _make_sc_gather(n_nodes, n_edges):
    ept = n_edges // NW
    ch = 6272
    assert ept % ch == 0 and ch % LANES == 0
    mesh = plsc.VectorSubcoreMesh(core_axis_name="c", subcore_axis_name="s")

    def body(h_hbm, esrc_hbm, edst_hbm, hs_hbm, hd_hbm,
             table, sbuf, dbuf, hsb, hdb):
        c = lax.axis_index("c")
        s = lax.axis_index("s")
        base = (s * NC + c) * ept
        pltpu.sync_copy(h_hbm, table)

        def chunk_body(ci, carry):
            off = base + ci * ch
            pltpu.sync_copy(esrc_hbm.at[pl.ds(off, ch)], sbuf)
            pltpu.sync_copy(edst_hbm.at[pl.ds(off, ch)], dbuf)

            def vec_body(k, carry2):
                i0 = k * LANES
                si = sbuf[pl.ds(i0, LANES)]
                di = dbuf[pl.ds(i0, LANES)]
                hsb[pl.ds(i0, LANES)] = plsc.load_gather(table, [si])
                hdb[pl.ds(i0, LANES)] = plsc.load_gather(table, [di])
                return carry2

            lax.fori_loop(0, ch // LANES, vec_body, 0, unroll=4)
            pltpu.sync_copy(hsb, hs_hbm.at[pl.ds(off, ch)])
            pltpu.sync_copy(hdb, hd_hbm.at[pl.ds(off, ch)])
            return carry

        lax.fori_loop(0, ept // ch, chunk_body, 0)

    return pl.kernel(
        body,
        out_type=[jax.ShapeDtypeStruct((n_edges,), jnp.float32),
                  jax.ShapeDtypeStruct((n_edges,), jnp.float32)],
        mesh=mesh,
        compiler_params=pltpu.CompilerParams(needs_layout_passes=False),
        scratch_types=[
            pltpu.VMEM((n_nodes,), jnp.float32),
            pltpu.VMEM((ch,), jnp.int32),
            pltpu.VMEM((ch,), jnp.int32),
            pltpu.VMEM((ch,), jnp.float32),
            pltpu.VMEM((ch,), jnp.float32),
        ],
    )


def _mlp_body(rc, w1_ref, b1_ref, w2_ref, b2_ref, hs_ref, hd_ref, w_ref,
              out_ref):
    hidden = w1_ref.shape[1]
    rows = hs_ref.shape[0]

    def chunk(k, carry):
        sl = pl.ds(k * rc, rc)
        hs = hs_ref[sl, :]
        hd = hd_ref[sl, :]
        w = w_ref[sl, :]

        def jb(j, acc):
            t = (hs * w1_ref[0, j] + hd * w1_ref[1, j] + w * w1_ref[2, j]
                 + b1_ref[j])
            return acc + jax.nn.gelu(t) * w2_ref[j]

        acc = lax.fori_loop(0, hidden, jb,
                            jnp.zeros((rc, 128), jnp.float32), unroll=4)
        out_ref[sl, :] = (acc + b2_ref[0]) * w
        return carry

    lax.fori_loop(0, rows // rc, chunk, 0)


def _make_tc_mlp(rows, hidden):
    rb = 448
    rc = 32
    assert rows % rb == 0 and rb % rc == 0
    grid = (rows // rb,)
    smem = functools.partial(pl.BlockSpec, memory_space=pltpu.SMEM)
    dblk = pl.BlockSpec((rb, 128), lambda i: (i, 0))
    return pl.pallas_call(
        functools.partial(_mlp_body, rc),
        grid=grid,
        in_specs=[smem((3, hidden), lambda i: (0, 0)),
                  smem((hidden,), lambda i: (0,)),
                  smem((hidden,), lambda i: (0,)),
                  smem((1,), lambda i: (0,)),
                  dblk, dblk, dblk],
        out_specs=dblk,
        out_shape=jax.ShapeDtypeStruct((rows, 128), jnp.float32),
    )


def _make_sc_scatter(n_nodes, rows):
    rpt = rows // NW
    kr = 56
    assert rpt % kr == 0
    nps = n_nodes // NS
    assert nps % LANES == 0
    mesh = plsc.VectorSubcoreMesh(core_axis_name="c", subcore_axis_name="s")

    def body(msg_hbm, edst_hbm, parts_hbm, idxb, valb, accl, acc_sh):
        c = lax.axis_index("c")
        s = lax.axis_index("s")
        wid = s * NC + c

        def zb(k, carry):
            accl[pl.ds(k * LANES, LANES)] = jnp.zeros((LANES,), jnp.float32)
            return carry

        lax.fori_loop(0, nps // LANES, zb, 0, unroll=8)
        pltpu.sync_copy(accl, acc_sh.at[pl.ds(s * nps, nps)])
        plsc.subcore_barrier()

        row0 = wid * rpt

        def chunk_body(ci, carry):
            r0 = row0 + ci * kr
            pltpu.sync_copy(edst_hbm.at[pl.ds(r0, kr), :], idxb)
            pltpu.sync_copy(msg_hbm.at[pl.ds(r0, kr), :], valb)

            def rower(j, carry2):
                pltpu.sync_copy(valb.at[j], acc_sh.at[idxb.at[j]], add=True)
                return carry2

            lax.fori_loop(0, kr, rower, 0)
            return carry

        lax.fori_loop(0, rpt // kr, chunk_body, 0)
        plsc.subcore_barrier()

        pltpu.sync_copy(acc_sh.at[pl.ds(s * nps, nps)], accl)
        pltpu.sync_copy(accl, parts_hbm.at[pl.ds(c * n_nodes + s * nps, nps)])

    return pl.kernel(
        body,
        out_type=jax.ShapeDtypeStruct((NC * n_nodes,), jnp.float32),
        mesh=mesh,
        compiler_params=pltpu.CompilerParams(needs_layout_passes=False),
        scratch_types=[
            pltpu.VMEM((kr, 128), jnp.int32),
            pltpu.VMEM((kr, 128), jnp.float32),
            pltpu.VMEM((nps,), jnp.float32),
            pltpu.VMEM_SHARED((n_nodes,), jnp.float32),
        ],
    )


def _combine_body(softplus, lnp_ref, h_ref, p_ref, deg_ref, out_ref):
    h = h_ref[...]
    agg = (p_ref[0] + p_ref[1]) / deg_ref[...]
    x = h + agg
    mu = x
    var = (x - mu) * (x - mu)
    hn = (x - mu) / jnp.sqrt(var + 1e-6) * lnp_ref[0] + lnp_ref[1]
    out_ref[...] = jax.nn.softplus(hn) if softplus else hn


def _make_tc_combine(nrows, softplus):
    full = pl.BlockSpec((nrows, 128), lambda: (0, 0))
    return pl.pallas_call(
        functools.partial(_combine_body, softplus),
        in_specs=[pl.BlockSpec(memory_space=pltpu.SMEM),
                  full,
                  pl.BlockSpec((2, nrows, 128), lambda: (0, 0, 0)),
                  full],
        out_specs=full,
        out_shape=jax.ShapeDtypeStruct((nrows, 128), jnp.float32),
    )


def kernel(eps_2d, esrc, edst, ew, ndeg, W1, b1, W2, b2, ln_scale, ln_bias):
    n_nodes = eps_2d.shape[0] * eps_2d.shape[1]
    n_edges = esrc.shape[0]
    n_layers, _, hidden = W1.shape
    rows = n_edges // 128
    nrows = n_nodes // 128

    h = eps_2d.reshape((n_nodes,))
    edst2d = edst.reshape((rows, 128))
    ew2d = ew.reshape((rows, 128))
    deg2d = ndeg.reshape((nrows, 128))

    sc_gather = _make_sc_gather(n_nodes, n_edges)
    tc_mlp = _make_tc_mlp(rows, hidden)
    sc_scatter = _make_sc_scatter(n_nodes, rows)

    for i in range(n_layers):
        hs, hd = sc_gather(h, esrc, edst)
        msgw = tc_mlp(W1[i], b1[i], W2[i, :, 0], b2[i],
                      hs.reshape((rows, 128)), hd.reshape((rows, 128)), ew2d)
        parts = sc_scatter(msgw, edst2d)
        lnp = jnp.stack([ln_scale[i, 0], ln_bias[i, 0]])
        combine = _make_tc_combine(nrows, softplus=(i == n_layers - 1))
        h2d = combine(lnp, h.reshape((nrows, 128)),
                      parts.reshape((2, nrows, 128)), deg2d)
        h = h2d.reshape((n_nodes,))

    return h.reshape(eps_2d.shape)


# speedup vs baseline: 1.4694x; 1.1078x over previous
import functools

import jax
import jax.numpy as jnp
from jax import lax
from jax.experimental import pallas as pl
from jax.experimental.pallas import tpu as pltpu
from jax.experimental.pallas import tpu_sc as plsc

NC = 2
NS = 16
NW = NC * NS
LANES = 16


def _make_sc_gather(n_nodes, n_edges):
    ept = n_edges // NW
    ch = 6272
    assert ept % ch == 0 and ch % LANES == 0
    mesh = plsc.VectorSubcoreMesh(core_axis_name="c", subcore_axis_name="s")

    def body(h_hbm, esrc_hbm, edst_hbm, hs_hbm, hd_hbm,
             table, sbuf, dbuf, hsb, hdb):
        c = lax.axis_index("c")
        s = lax.axis_index("s")
        base = (s * NC + c) * ept
        pltpu.sync_copy(h_hbm, table)

        def chunk_body(ci, carry):
            off = base + ci * ch
            pltpu.sync_copy(esrc_hbm.at[pl.ds(off, ch)], sbuf)
            pltpu.sync_copy(edst_hbm.at[pl.ds(off, ch)], dbuf)

            def vec_body(k, carry2):
                i0 = k * LANES
                si = sbuf[pl.ds(i0, LANES)]
                di = dbuf[pl.ds(i0, LANES)]
                hsb[pl.ds(i0, LANES)] = plsc.load_gather(table, [si])
                hdb[pl.ds(i0, LANES)] = plsc.load_gather(table, [di])
                return carry2

            lax.fori_loop(0, ch // LANES, vec_body, 0, unroll=4)
            pltpu.sync_copy(hsb, hs_hbm.at[pl.ds(off, ch)])
            pltpu.sync_copy(hdb, hd_hbm.at[pl.ds(off, ch)])
            return carry

        lax.fori_loop(0, ept // ch, chunk_body, 0)

    return pl.kernel(
        body,
        out_type=[jax.ShapeDtypeStruct((n_edges,), jnp.float32),
                  jax.ShapeDtypeStruct((n_edges,), jnp.float32)],
        mesh=mesh,
        compiler_params=pltpu.CompilerParams(needs_layout_passes=False),
        scratch_types=[
            pltpu.VMEM((n_nodes,), jnp.float32),
            pltpu.VMEM((ch,), jnp.int32),
            pltpu.VMEM((ch,), jnp.int32),
            pltpu.VMEM((ch,), jnp.float32),
            pltpu.VMEM((ch,), jnp.float32),
        ],
    )


def _mlp_body(rc, w1_ref, b1_ref, w2_ref, b2_ref, hs_ref, hd_ref, w_ref,
              out_ref):
    hidden = w1_ref.shape[1]
    rows = hs_ref.shape[0]

    def chunk(k, carry):
        sl = pl.ds(k * rc, rc)
        hs = hs_ref[sl, :]
        hd = hd_ref[sl, :]
        w = w_ref[sl, :]

        def jb(j, acc):
            t = (hs * w1_ref[0, j] + hd * w1_ref[1, j] + w * w1_ref[2, j]
                 + b1_ref[j])
            return acc + jax.nn.gelu(t) * w2_ref[j]

        acc = lax.fori_loop(0, hidden, jb,
                            jnp.zeros((rc, 128), jnp.float32), unroll=8)
        out_ref[sl, :] = (acc + b2_ref[0]) * w
        return carry

    lax.fori_loop(0, rows // rc, chunk, 0)


def _make_tc_mlp(rows, hidden):
    rb = 448
    rc = 32
    assert rows % rb == 0 and rb % rc == 0
    grid = (rows // rb,)
    smem = functools.partial(pl.BlockSpec, memory_space=pltpu.SMEM)
    dblk = pl.BlockSpec((rb, 128), lambda i: (i, 0))
    return pl.pallas_call(
        functools.partial(_mlp_body, rc),
        grid=grid,
        in_specs=[smem((3, hidden), lambda i: (0, 0)),
                  smem((hidden,), lambda i: (0,)),
                  smem((hidden,), lambda i: (0,)),
                  smem((1,), lambda i: (0,)),
                  dblk, dblk, dblk],
        out_specs=dblk,
        out_shape=jax.ShapeDtypeStruct((rows, 128), jnp.float32),
    )


def _make_sc_scatter(n_nodes, rows):
    rpt = rows // NW
    kr = 56
    assert rpt % kr == 0
    nps = n_nodes // NS
    assert nps % LANES == 0
    mesh = plsc.VectorSubcoreMesh(core_axis_name="c", subcore_axis_name="s")

    def body(msg_hbm, edst_hbm, parts_hbm, idxb, valb, accl, acc_sh):
        c = lax.axis_index("c")
        s = lax.axis_index("s")
        wid = s * NC + c

        def zb(k, carry):
            accl[pl.ds(k * LANES, LANES)] = jnp.zeros((LANES,), jnp.float32)
            return carry

        lax.fori_loop(0, nps // LANES, zb, 0, unroll=8)
        pltpu.sync_copy(accl, acc_sh.at[pl.ds(s * nps, nps)])
        plsc.subcore_barrier()

        row0 = wid * rpt

        def chunk_body(ci, carry):
            r0 = row0 + ci * kr
            pltpu.sync_copy(edst_hbm.at[pl.ds(r0, kr), :], idxb)
            pltpu.sync_copy(msg_hbm.at[pl.ds(r0, kr), :], valb)

            def rower(j, carry2):
                pltpu.sync_copy(valb.at[j], acc_sh.at[idxb.at[j]], add=True)
                return carry2

            lax.fori_loop(0, kr, rower, 0)
            return carry

        lax.fori_loop(0, rpt // kr, chunk_body, 0)
        plsc.subcore_barrier()

        pltpu.sync_copy(acc_sh.at[pl.ds(s * nps, nps)], accl)
        pltpu.sync_copy(accl, parts_hbm.at[pl.ds(c * n_nodes + s * nps, nps)])

    return pl.kernel(
        body,
        out_type=jax.ShapeDtypeStruct((NC * n_nodes,), jnp.float32),
        mesh=mesh,
        compiler_params=pltpu.CompilerParams(needs_layout_passes=False),
        scratch_types=[
            pltpu.VMEM((kr, 128), jnp.int32),
            pltpu.VMEM((kr, 128), jnp.float32),
            pltpu.VMEM((nps,), jnp.float32),
            pltpu.VMEM_SHARED((n_nodes,), jnp.float32),
        ],
    )


def _combine_body(softplus, lnp_ref, h_ref, p_ref, deg_ref, out_ref):
    h = h_ref[...]
    agg = (p_ref[0] + p_ref[1]) / deg_ref[...]
    x = h + agg
    mu = x
    var = (x - mu) * (x - mu)
    hn = (x - mu) / jnp.sqrt(var + 1e-6) * lnp_ref[0] + lnp_ref[1]
    out_ref[...] = jax.nn.softplus(hn) if softplus else hn


def _make_tc_combine(nrows, softplus):
    full = pl.BlockSpec((nrows, 128), lambda: (0, 0))
    return pl.pallas_call(
        functools.partial(_combine_body, softplus),
        in_specs=[pl.BlockSpec(memory_space=pltpu.SMEM),
                  full,
                  pl.BlockSpec((2, nrows, 128), lambda: (0, 0, 0)),
                  full],
        out_specs=full,
        out_shape=jax.ShapeDtypeStruct((nrows, 128), jnp.float32),
    )


def kernel(eps_2d, esrc, edst, ew, ndeg, W1, b1, W2, b2, ln_scale, ln_bias):
    n_nodes = eps_2d.shape[0] * eps_2d.shape[1]
    n_edges = esrc.shape[0]
    n_layers, _, hidden = W1.shape
    rows = n_edges // 128
    nrows = n_nodes // 128

    h = eps_2d.reshape((n_nodes,))
    edst2d = edst.reshape((rows, 128))
    ew2d = ew.reshape((rows, 128))
    deg2d = ndeg.reshape((nrows, 128))

    sc_gather = _make_sc_gather(n_nodes, n_edges)
    tc_mlp = _make_tc_mlp(rows, hidden)
    sc_scatter = _make_sc_scatter(n_nodes, rows)

    for i in range(n_layers):
        hs, hd = sc_gather(h, esrc, edst)
        msgw = tc_mlp(W1[i], b1[i], W2[i, :, 0], b2[i],
                      hs.reshape((rows, 128)), hd.reshape((rows, 128)), ew2d)
        parts = sc_scatter(msgw, edst2d)
        lnp = jnp.stack([ln_scale[i, 0], ln_bias[i, 0]])
        combine = _make_tc_combine(nrows, softplus=(i == n_layers - 1))
        h2d = combine(lnp, h.reshape((nrows, 128)),
                      parts.reshape((2, nrows, 128)), deg2d)
        h = h2d.reshape((n_nodes,))

    return h.reshape(eps_2d.shape)


# speedup vs baseline: 1.5281x; 1.0399x over previous
import functools

import jax
import jax.numpy as jnp
from jax import lax
from jax.experimental import pallas as pl
from jax.experimental.pallas import tpu as pltpu
from jax.experimental.pallas import tpu_sc as plsc

NC = 2
NS = 16
NW = NC * NS
LANES = 16


def _make_sc_gather(n_nodes, n_edges):
    ept = n_edges // NW
    ch = 6272
    assert ept % ch == 0 and ch % LANES == 0
    mesh = plsc.VectorSubcoreMesh(core_axis_name="c", subcore_axis_name="s")

    def body(h_hbm, esrc_hbm, edst_hbm, hs_hbm, hd_hbm,
             table, sbuf, dbuf, hsb, hdb):
        c = lax.axis_index("c")
        s = lax.axis_index("s")
        base = (s * NC + c) * ept
        pltpu.sync_copy(h_hbm, table)

        def chunk_body(ci, carry):
            off = base + ci * ch
            pltpu.sync_copy(esrc_hbm.at[pl.ds(off, ch)], sbuf)
            pltpu.sync_copy(edst_hbm.at[pl.ds(off, ch)], dbuf)

            def vec_body(k, carry2):
                i0 = k * LANES
                si = sbuf[pl.ds(i0, LANES)]
                di = dbuf[pl.ds(i0, LANES)]
                hsb[pl.ds(i0, LANES)] = plsc.load_gather(table, [si])
                hdb[pl.ds(i0, LANES)] = plsc.load_gather(table, [di])
                return carry2

            lax.fori_loop(0, ch // LANES, vec_body, 0, unroll=4)
            pltpu.sync_copy(hsb, hs_hbm.at[pl.ds(off, ch)])
            pltpu.sync_copy(hdb, hd_hbm.at[pl.ds(off, ch)])
            return carry

        lax.fori_loop(0, ept // ch, chunk_body, 0)

    return pl.kernel(
        body,
        out_type=[jax.ShapeDtypeStruct((n_edges,), jnp.float32),
                  jax.ShapeDtypeStruct((n_edges,), jnp.float32)],
        mesh=mesh,
        compiler_params=pltpu.CompilerParams(needs_layout_passes=False),
        scratch_types=[
            pltpu.VMEM((n_nodes,), jnp.float32),
            pltpu.VMEM((ch,), jnp.int32),
            pltpu.VMEM((ch,), jnp.int32),
            pltpu.VMEM((ch,), jnp.float32),
            pltpu.VMEM((ch,), jnp.float32),
        ],
    )


def _mlp_body(rc, w1_ref, b1_ref, w2_ref, b2_ref, hs_ref, hd_ref, w_ref,
              out_ref):
    hidden = w1_ref.shape[1]
    rows = hs_ref.shape[0]

    def chunk(k, carry):
        sl = pl.ds(k * rc, rc)
        hs = hs_ref[sl, :]
        hd = hd_ref[sl, :]
        w = w_ref[sl, :]

        def jb(j, acc):
            t = (hs * w1_ref[0, j] + hd * w1_ref[1, j] + w * w1_ref[2, j]
                 + b1_ref[j])
            return acc + jax.nn.gelu(t) * w2_ref[j]

        acc = lax.fori_loop(0, hidden, jb,
                            jnp.zeros((rc, 128), jnp.float32), unroll=4)
        out_ref[sl, :] = (acc + b2_ref[0]) * w
        return carry

    lax.fori_loop(0, rows // rc, chunk, 0)


def _make_tc_mlp(rows, hidden):
    rb = 448
    rc = 64
    assert rows % rb == 0 and rb % rc == 0
    grid = (rows // rb,)
    smem = functools.partial(pl.BlockSpec, memory_space=pltpu.SMEM)
    dblk = pl.BlockSpec((rb, 128), lambda i: (i, 0))
    return pl.pallas_call(
        functools.partial(_mlp_body, rc),
        grid=grid,
        in_specs=[smem((3, hidden), lambda i: (0, 0)),
                  smem((hidden,), lambda i: (0,)),
                  smem((hidden,), lambda i: (0,)),
                  smem((1,), lambda i: (0,)),
                  dblk, dblk, dblk],
        out_specs=dblk,
        out_shape=jax.ShapeDtypeStruct((rows, 128), jnp.float32),
    )


def _make_sc_scatter(n_nodes, rows):
    rpt = rows // NW
    kr = 56
    assert rpt % kr == 0
    nps = n_nodes // NS
    assert nps % LANES == 0
    mesh = plsc.VectorSubcoreMesh(core_axis_name="c", subcore_axis_name="s")

    def body(msg_hbm, edst_hbm, parts_hbm, idxb, valb, accl, acc_sh):
        c = lax.axis_index("c")
        s = lax.axis_index("s")
        wid = s * NC + c

        def zb(k, carry):
            accl[pl.ds(k * LANES, LANES)] = jnp.zeros((LANES,), jnp.float32)
            return carry

        lax.fori_loop(0, nps // LANES, zb, 0, unroll=8)
        pltpu.sync_copy(accl, acc_sh.at[pl.ds(s * nps, nps)])
        plsc.subcore_barrier()

        row0 = wid * rpt

        def chunk_body(ci, carry):
            r0 = row0 + ci * kr
            pltpu.sync_copy(edst_hbm.at[pl.ds(r0, kr), :], idxb)
            pltpu.sync_copy(msg_hbm.at[pl.ds(r0, kr), :], valb)

            def rower(j, carry2):
                pltpu.sync_copy(valb.at[j], acc_sh.at[idxb.at[j]], add=True)
                return carry2

            lax.fori_loop(0, kr, rower, 0)
            return carry

        lax.fori_loop(0, rpt // kr, chunk_body, 0)
        plsc.subcore_barrier()

        pltpu.sync_copy(acc_sh.at[pl.ds(s * nps, nps)], accl)
        pltpu.sync_copy(accl, parts_hbm.at[pl.ds(c * n_nodes + s * nps, nps)])

    return pl.kernel(
        body,
        out_type=jax.ShapeDtypeStruct((NC * n_nodes,), jnp.float32),
        mesh=mesh,
        compiler_params=pltpu.CompilerParams(needs_layout_passes=False),
        scratch_types=[
            pltpu.VMEM((kr, 128), jnp.int32),
            pltpu.VMEM((kr, 128), jnp.float32),
            pltpu.VMEM((nps,), jnp.float32),
            pltpu.VMEM_SHARED((n_nodes,), jnp.float32),
        ],
    )


def _combine_body(softplus, lnp_ref, h_ref, p_ref, deg_ref, out_ref):
    h = h_ref[...]
    agg = (p_ref[0] + p_ref[1]) / deg_ref[...]
    x = h + agg
    mu = x
    var = (x - mu) * (x - mu)
    hn = (x - mu) / jnp.sqrt(var + 1e-6) * lnp_ref[0] + lnp_ref[1]
    out_ref[...] = jax.nn.softplus(hn) if softplus else hn


def _make_tc_combine(nrows, softplus):
    full = pl.BlockSpec((nrows, 128), lambda: (0, 0))
    return pl.pallas_call(
        functools.partial(_combine_body, softplus),
        in_specs=[pl.BlockSpec(memory_space=pltpu.SMEM),
                  full,
                  pl.BlockSpec((2, nrows, 128), lambda: (0, 0, 0)),
                  full],
        out_specs=full,
        out_shape=jax.ShapeDtypeStruct((nrows, 128), jnp.float32),
    )


def kernel(eps_2d, esrc, edst, ew, ndeg, W1, b1, W2, b2, ln_scale, ln_bias):
    n_nodes = eps_2d.shape[0] * eps_2d.shape[1]
    n_edges = esrc.shape[0]
    n_layers, _, hidden = W1.shape
    rows = n_edges // 128
    nrows = n_nodes // 128

    h = eps_2d.reshape((n_nodes,))
    edst2d = edst.reshape((rows, 128))
    ew2d = ew.reshape((rows, 128))
    deg2d = ndeg.reshape((nrows, 128))

    sc_gather = _make_sc_gather(n_nodes, n_edges)
    tc_mlp = _make_tc_mlp(rows, hidden)
    sc_scatter = _make_sc_scatter(n_nodes, rows)

    for i in range(n_layers):
        hs, hd = sc_gather(h, esrc, edst)
        msgw = tc_mlp(W1[i], b1[i], W2[i, :, 0], b2[i],
                      hs.reshape((rows, 128)), hd.reshape((rows, 128)), ew2d)
        parts = sc_scatter(msgw, edst2d)
        lnp = jnp.stack([ln_scale[i, 0], ln_bias[i, 0]])
        combine = _make_tc_combine(nrows, softplus=(i == n_layers - 1))
        h2d = combine(lnp, h.reshape((nrows, 128)),
                      parts.reshape((2, nrows, 128)), deg2d)
        h = h2d.reshape((n_nodes,))

    return h.reshape(eps_2d.shape)


# speedup vs baseline: 1.5879x; 1.0392x over previous
import functools

import jax
import jax.numpy as jnp
from jax import lax
from jax.experimental import pallas as pl
from jax.experimental.pallas import tpu as pltpu
from jax.experimental.pallas import tpu_sc as plsc

NC = 2
NS = 16
NW = NC * NS
LANES = 16


def _make_sc_gather(n_nodes, n_edges):
    ept = n_edges // NW
    ch = 6272
    assert ept % ch == 0 and ch % LANES == 0
    mesh = plsc.VectorSubcoreMesh(core_axis_name="c", subcore_axis_name="s")

    def body(h_hbm, esrc_hbm, edst_hbm, hs_hbm, hd_hbm,
             table, sbuf, dbuf, hsb, hdb):
        c = lax.axis_index("c")
        s = lax.axis_index("s")
        base = (s * NC + c) * ept
        pltpu.sync_copy(h_hbm, table)

        def chunk_body(ci, carry):
            off = base + ci * ch
            pltpu.sync_copy(esrc_hbm.at[pl.ds(off, ch)], sbuf)
            pltpu.sync_copy(edst_hbm.at[pl.ds(off, ch)], dbuf)

            def vec_body(k, carry2):
                i0 = k * LANES
                si = sbuf[pl.ds(i0, LANES)]
                di = dbuf[pl.ds(i0, LANES)]
                hsb[pl.ds(i0, LANES)] = plsc.load_gather(table, [si])
                hdb[pl.ds(i0, LANES)] = plsc.load_gather(table, [di])
                return carry2

            lax.fori_loop(0, ch // LANES, vec_body, 0, unroll=4)
            pltpu.sync_copy(hsb, hs_hbm.at[pl.ds(off, ch)])
            pltpu.sync_copy(hdb, hd_hbm.at[pl.ds(off, ch)])
            return carry

        lax.fori_loop(0, ept // ch, chunk_body, 0)

    return pl.kernel(
        body,
        out_type=[jax.ShapeDtypeStruct((n_edges,), jnp.float32),
                  jax.ShapeDtypeStruct((n_edges,), jnp.float32)],
        mesh=mesh,
        compiler_params=pltpu.CompilerParams(needs_layout_passes=False),
        scratch_types=[
            pltpu.VMEM((n_nodes,), jnp.float32),
            pltpu.VMEM((ch,), jnp.int32),
            pltpu.VMEM((ch,), jnp.int32),
            pltpu.VMEM((ch,), jnp.float32),
            pltpu.VMEM((ch,), jnp.float32),
        ],
    )


def _mlp_body(rc, w1_ref, b1_ref, w2_ref, b2_ref, hs_ref, hd_ref, w_ref,
              out_ref):
    hidden = w1_ref.shape[1]
    rows = hs_ref.shape[0]

    def chunk(k, carry):
        sl = pl.ds(k * rc, rc)
        hs = hs_ref[sl, :]
        hd = hd_ref[sl, :]
        w = w_ref[sl, :]

        def jb(j, acc):
            t = (hs * w1_ref[0, j] + hd * w1_ref[1, j] + w * w1_ref[2, j]
                 + b1_ref[j])
            return acc + jax.nn.gelu(t) * w2_ref[j]

        acc = lax.fori_loop(0, hidden, jb,
                            jnp.zeros((rc, 128), jnp.float32), unroll=8)
        out_ref[sl, :] = (acc + b2_ref[0]) * w
        return carry

    lax.fori_loop(0, rows // rc, chunk, 0)


def _make_tc_mlp(rows, hidden):
    rb = 448
    rc = 64
    assert rows % rb == 0 and rb % rc == 0
    grid = (rows // rb,)
    smem = functools.partial(pl.BlockSpec, memory_space=pltpu.SMEM)
    dblk = pl.BlockSpec((rb, 128), lambda i: (i, 0))
    return pl.pallas_call(
        functools.partial(_mlp_body, rc),
        grid=grid,
        in_specs=[smem((3, hidden), lambda i: (0, 0)),
                  smem((hidden,), lambda i: (0,)),
                  smem((hidden,), lambda i: (0,)),
                  smem((1,), lambda i: (0,)),
                  dblk, dblk, dblk],
        out_specs=dblk,
        out_shape=jax.ShapeDtypeStruct((rows, 128), jnp.float32),
    )


def _make_sc_scatter(n_nodes, rows):
    rpt = rows // NW
    kr = 56
    assert rpt % kr == 0
    nps = n_nodes // NS
    assert nps % LANES == 0
    mesh = plsc.VectorSubcoreMesh(core_axis_name="c", subcore_axis_name="s")

    def body(msg_hbm, edst_hbm, parts_hbm, idxb, valb, accl, acc_sh):
        c = lax.axis_index("c")
        s = lax.axis_index("s")
        wid = s * NC + c

        def zb(k, carry):
            accl[pl.ds(k * LANES, LANES)] = jnp.zeros((LANES,), jnp.float32)
            return carry

        lax.fori_loop(0, nps // LANES, zb, 0, unroll=8)
        pltpu.sync_copy(accl, acc_sh.at[pl.ds(s * nps, nps)])
        plsc.subcore_barrier()

        row0 = wid * rpt

        def chunk_body(ci, carry):
            r0 = row0 + ci * kr
            pltpu.sync_copy(edst_hbm.at[pl.ds(r0, kr), :], idxb)
            pltpu.sync_copy(msg_hbm.at[pl.ds(r0, kr), :], valb)

            def rower(j, carry2):
                pltpu.sync_copy(valb.at[j], acc_sh.at[idxb.at[j]], add=True)
                return carry2

            lax.fori_loop(0, kr, rower, 0)
            return carry

        lax.fori_loop(0, rpt // kr, chunk_body, 0)
        plsc.subcore_barrier()

        pltpu.sync_copy(acc_sh.at[pl.ds(s * nps, nps)], accl)
        pltpu.sync_copy(accl, parts_hbm.at[pl.ds(c * n_nodes + s * nps, nps)])

    return pl.kernel(
        body,
        out_type=jax.ShapeDtypeStruct((NC * n_nodes,), jnp.float32),
        mesh=mesh,
        compiler_params=pltpu.CompilerParams(needs_layout_passes=False),
        scratch_types=[
            pltpu.VMEM((kr, 128), jnp.int32),
            pltpu.VMEM((kr, 128), jnp.float32),
            pltpu.VMEM((nps,), jnp.float32),
            pltpu.VMEM_SHARED((n_nodes,), jnp.float32),
        ],
    )


def _combine_body(softplus, lnp_ref, h_ref, p_ref, deg_ref, out_ref):
    h = h_ref[...]
    agg = (p_ref[0] + p_ref[1]) / deg_ref[...]
    x = h + agg
    mu = x
    var = (x - mu) * (x - mu)
    hn = (x - mu) / jnp.sqrt(var + 1e-6) * lnp_ref[0] + lnp_ref[1]
    out_ref[...] = jax.nn.softplus(hn) if softplus else hn


def _make_tc_combine(nrows, softplus):
    full = pl.BlockSpec((nrows, 128), lambda: (0, 0))
    return pl.pallas_call(
        functools.partial(_combine_body, softplus),
        in_specs=[pl.BlockSpec(memory_space=pltpu.SMEM),
                  full,
                  pl.BlockSpec((2, nrows, 128), lambda: (0, 0, 0)),
                  full],
        out_specs=full,
        out_shape=jax.ShapeDtypeStruct((nrows, 128), jnp.float32),
    )


def kernel(eps_2d, esrc, edst, ew, ndeg, W1, b1, W2, b2, ln_scale, ln_bias):
    n_nodes = eps_2d.shape[0] * eps_2d.shape[1]
    n_edges = esrc.shape[0]
    n_layers, _, hidden = W1.shape
    rows = n_edges // 128
    nrows = n_nodes // 128

    h = eps_2d.reshape((n_nodes,))
    edst2d = edst.reshape((rows, 128))
    ew2d = ew.reshape((rows, 128))
    deg2d = ndeg.reshape((nrows, 128))

    sc_gather = _make_sc_gather(n_nodes, n_edges)
    tc_mlp = _make_tc_mlp(rows, hidden)
    sc_scatter = _make_sc_scatter(n_nodes, rows)

    for i in range(n_layers):
        hs, hd = sc_gather(h, esrc, edst)
        msgw = tc_mlp(W1[i], b1[i], W2[i, :, 0], b2[i],
                      hs.reshape((rows, 128)), hd.reshape((rows, 128)), ew2d)
        parts = sc_scatter(msgw, edst2d)
        lnp = jnp.stack([ln_scale[i, 0], ln_bias[i, 0]])
        combine = _make_tc_combine(nrows, softplus=(i == n_layers - 1))
        h2d = combine(lnp, h.reshape((nrows, 128)),
                      parts.reshape((2, nrows, 128)), deg2d)
        h = h2d.reshape((n_nodes,))

    return h.reshape(eps_2d.shape)


# speedup vs baseline: 1.6192x; 1.0197x over previous
import functools

import jax
import jax.numpy as jnp
from jax import lax
from jax.experimental import pallas as pl
from jax.experimental.pallas import tpu as pltpu
from jax.experimental.pallas import tpu_sc as plsc

NC = 2
NS = 16
NW = NC * NS
LANES = 16


def _make_sc_gather(n_nodes, n_edges):
    ept = n_edges // NW
    ch = 6272
    assert ept % ch == 0 and ch % LANES == 0
    mesh = plsc.VectorSubcoreMesh(core_axis_name="c", subcore_axis_name="s")

    def body(h_hbm, esrc_hbm, edst_hbm, hs_hbm, hd_hbm,
             table, sbuf, dbuf, hsb, hdb):
        c = lax.axis_index("c")
        s = lax.axis_index("s")
        base = (s * NC + c) * ept
        pltpu.sync_copy(h_hbm, table)

        def chunk_body(ci, carry):
            off = base + ci * ch
            pltpu.sync_copy(esrc_hbm.at[pl.ds(off, ch)], sbuf)
            pltpu.sync_copy(edst_hbm.at[pl.ds(off, ch)], dbuf)

            def vec_body(k, carry2):
                i0 = k * LANES
                si = sbuf[pl.ds(i0, LANES)]
                di = dbuf[pl.ds(i0, LANES)]
                hsb[pl.ds(i0, LANES)] = plsc.load_gather(table, [si])
                hdb[pl.ds(i0, LANES)] = plsc.load_gather(table, [di])
                return carry2

            lax.fori_loop(0, ch // LANES, vec_body, 0, unroll=4)
            pltpu.sync_copy(hsb, hs_hbm.at[pl.ds(off, ch)])
            pltpu.sync_copy(hdb, hd_hbm.at[pl.ds(off, ch)])
            return carry

        lax.fori_loop(0, ept // ch, chunk_body, 0)

    return pl.kernel(
        body,
        out_type=[jax.ShapeDtypeStruct((n_edges,), jnp.float32),
                  jax.ShapeDtypeStruct((n_edges,), jnp.float32)],
        mesh=mesh,
        compiler_params=pltpu.CompilerParams(needs_layout_passes=False),
        scratch_types=[
            pltpu.VMEM((n_nodes,), jnp.float32),
            pltpu.VMEM((ch,), jnp.int32),
            pltpu.VMEM((ch,), jnp.int32),
            pltpu.VMEM((ch,), jnp.float32),
            pltpu.VMEM((ch,), jnp.float32),
        ],
    )


def _mlp_body(rc, w1_ref, b1_ref, w2_ref, b2_ref, hs_ref, hd_ref, w_ref,
              out_ref):
    hidden = w1_ref.shape[1]
    rows = hs_ref.shape[0]

    def chunk(k, carry):
        sl = pl.ds(k * rc, rc)
        hs = hs_ref[sl, :]
        hd = hd_ref[sl, :]
        w = w_ref[sl, :]

        def jb(j, acc):
            t = (hs * w1_ref[0, j] + hd * w1_ref[1, j] + w * w1_ref[2, j]
                 + b1_ref[j])
            return acc + jax.nn.gelu(t) * w2_ref[j]

        acc = lax.fori_loop(0, hidden, jb,
                            jnp.zeros((rc, 128), jnp.float32), unroll=16)
        out_ref[sl, :] = (acc + b2_ref[0]) * w
        return carry

    lax.fori_loop(0, rows // rc, chunk, 0)


def _make_tc_mlp(rows, hidden):
    rb = 448
    rc = 64
    assert rows % rb == 0 and rb % rc == 0
    grid = (rows // rb,)
    smem = functools.partial(pl.BlockSpec, memory_space=pltpu.SMEM)
    dblk = pl.BlockSpec((rb, 128), lambda i: (i, 0))
    return pl.pallas_call(
        functools.partial(_mlp_body, rc),
        grid=grid,
        in_specs=[smem((3, hidden), lambda i: (0, 0)),
                  smem((hidden,), lambda i: (0,)),
                  smem((hidden,), lambda i: (0,)),
                  smem((1,), lambda i: (0,)),
                  dblk, dblk, dblk],
        out_specs=dblk,
        out_shape=jax.ShapeDtypeStruct((rows, 128), jnp.float32),
    )


def _make_sc_scatter(n_nodes, rows):
    rpt = rows // NW
    kr = 56
    assert rpt % kr == 0
    nps = n_nodes // NS
    assert nps % LANES == 0
    mesh = plsc.VectorSubcoreMesh(core_axis_name="c", subcore_axis_name="s")

    def body(msg_hbm, edst_hbm, parts_hbm, idxb, valb, accl, acc_sh):
        c = lax.axis_index("c")
        s = lax.axis_index("s")
        wid = s * NC + c

        def zb(k, carry):
            accl[pl.ds(k * LANES, LANES)] = jnp.zeros((LANES,), jnp.float32)
            return carry

        lax.fori_loop(0, nps // LANES, zb, 0, unroll=8)
        pltpu.sync_copy(accl, acc_sh.at[pl.ds(s * nps, nps)])
        plsc.subcore_barrier()

        row0 = wid * rpt

        def chunk_body(ci, carry):
            r0 = row0 + ci * kr
            pltpu.sync_copy(edst_hbm.at[pl.ds(r0, kr), :], idxb)
            pltpu.sync_copy(msg_hbm.at[pl.ds(r0, kr), :], valb)

            def rower(j, carry2):
                pltpu.sync_copy(valb.at[j], acc_sh.at[idxb.at[j]], add=True)
                return carry2

            lax.fori_loop(0, kr, rower, 0)
            return carry

        lax.fori_loop(0, rpt // kr, chunk_body, 0)
        plsc.subcore_barrier()

        pltpu.sync_copy(acc_sh.at[pl.ds(s * nps, nps)], accl)
        pltpu.sync_copy(accl, parts_hbm.at[pl.ds(c * n_nodes + s * nps, nps)])

    return pl.kernel(
        body,
        out_type=jax.ShapeDtypeStruct((NC * n_nodes,), jnp.float32),
        mesh=mesh,
        compiler_params=pltpu.CompilerParams(needs_layout_passes=False),
        scratch_types=[
            pltpu.VMEM((kr, 128), jnp.int32),
            pltpu.VMEM((kr, 128), jnp.float32),
            pltpu.VMEM((nps,), jnp.float32),
            pltpu.VMEM_SHARED((n_nodes,), jnp.float32),
        ],
    )


def _combine_body(softplus, lnp_ref, h_ref, p_ref, deg_ref, out_ref):
    h = h_ref[...]
    agg = (p_ref[0] + p_ref[1]) / deg_ref[...]
    x = h + agg
    mu = x
    var = (x - mu) * (x - mu)
    hn = (x - mu) / jnp.sqrt(var + 1e-6) * lnp_ref[0] + lnp_ref[1]
    out_ref[...] = jax.nn.softplus(hn) if softplus else hn


def _make_tc_combine(nrows, softplus):
    full = pl.BlockSpec((nrows, 128), lambda: (0, 0))
    return pl.pallas_call(
        functools.partial(_combine_body, softplus),
        in_specs=[pl.BlockSpec(memory_space=pltpu.SMEM),
                  full,
                  pl.BlockSpec((2, nrows, 128), lambda: (0, 0, 0)),
                  full],
        out_specs=full,
        out_shape=jax.ShapeDtypeStruct((nrows, 128), jnp.float32),
    )


def kernel(eps_2d, esrc, edst, ew, ndeg, W1, b1, W2, b2, ln_scale, ln_bias):
    n_nodes = eps_2d.shape[0] * eps_2d.shape[1]
    n_edges = esrc.shape[0]
    n_layers, _, hidden = W1.shape
    rows = n_edges // 128
    nrows = n_nodes // 128

    h = eps_2d.reshape((n_nodes,))
    edst2d = edst.reshape((rows, 128))
    ew2d = ew.reshape((rows, 128))
    deg2d = ndeg.reshape((nrows, 128))

    sc_gather = _make_sc_gather(n_nodes, n_edges)
    tc_mlp = _make_tc_mlp(rows, hidden)
    sc_scatter = _make_sc_scatter(n_nodes, rows)

    for i in range(n_layers):
        hs, hd = sc_gather(h, esrc, edst)
        msgw = tc_mlp(W1[i], b1[i], W2[i, :, 0], b2[i],
                      hs.reshape((rows, 128)), hd.reshape((rows, 128)), ew2d)
        parts = sc_scatter(msgw, edst2d)
        lnp = jnp.stack([ln_scale[i, 0], ln_bias[i, 0]])
        combine = _make_tc_combine(nrows, softplus=(i == n_layers - 1))
        h2d = combine(lnp, h.reshape((nrows, 128)),
                      parts.reshape((2, nrows, 128)), deg2d)
        h = h2d.reshape((n_nodes,))

    return h.reshape(eps_2d.shape)
